# Initial kernel scaffold; baseline (speedup 1.0000x reference)
import jax
import jax.numpy as jnp
from jax.experimental import pallas as pl


def kernel(encoded_parents, masks, kmer_emb, site_emb):
    raise NotImplementedError("write your pallas kernel here")


# speedup vs baseline: 315.2509x; 315.2509x over previous
import functools

import jax
import jax.numpy as jnp
from jax import lax
from jax.experimental import pallas as pl
from jax.experimental.pallas import tpu as pltpu
from jax.experimental.pallas import tpu_sc as plsc

BATCH = 16384
SEQ = 500
KMER = 1024
L = 16
NC = 2
NS = 16
NW = NC * NS
ROWS_PER_W = BATCH // NW
RBLK = 64
NBLK = ROWS_PER_W // RBLK
OFFS = tuple(16 * j for j in range(SEQ // L)) + (SEQ - L,)


@functools.partial(
    pl.kernel,
    out_type=jax.ShapeDtypeStruct((BATCH, SEQ), jnp.float32),
    mesh=plsc.VectorSubcoreMesh(core_axis_name="c", subcore_axis_name="s"),
    compiler_params=pltpu.CompilerParams(needs_layout_passes=False),
    scratch_types=[
        pltpu.VMEM((KMER,), jnp.float32),
        pltpu.VMEM((512,), jnp.float32),
        pltpu.VMEM((RBLK, SEQ), jnp.int32),
        pltpu.VMEM((RBLK, SEQ), jnp.float32),
    ],
)
def _sc_rates(parents_hbm, ktab_hbm, stab_hbm, out_hbm,
              ket_v, set_v, pin_v, pout_v):
    wid = lax.axis_index("s") * NC + lax.axis_index("c")
    base = wid * ROWS_PER_W

    zeros = jnp.zeros((L,), jnp.float32)
    set_v[pl.ds(480, L)] = zeros
    set_v[pl.ds(496, L)] = zeros
    pltpu.sync_copy(ktab_hbm, ket_v)
    pltpu.sync_copy(stab_hbm, set_v.at[pl.ds(0, SEQ)])
    for j in range(KMER // L):
        ket_v[pl.ds(j * L, L)] = jnp.exp(ket_v[pl.ds(j * L, L)])
    for j in range(512 // L):
        set_v[pl.ds(j * L, L)] = jnp.exp(set_v[pl.ds(j * L, L)])

    for g in range(NBLK):
        row0 = base + g * RBLK
        pltpu.sync_copy(parents_hbm.at[pl.ds(row0, RBLK), :], pin_v)
        for off in OFFS:
            sv = set_v[pl.ds(off, L)]

            def row_body(r, carry, *, _off=off, _sv=sv):
                idx = pin_v[r, pl.ds(_off, L)]
                vals = plsc.load_gather(ket_v, [idx])
                pout_v[r, pl.ds(_off, L)] = vals * _sv
                return carry

            lax.fori_loop(0, RBLK, row_body, 0)
        pltpu.sync_copy(pout_v, out_hbm.at[pl.ds(row0, RBLK), :])


def kernel(encoded_parents, masks, kmer_emb, site_emb):
    del masks
    out = _sc_rates(encoded_parents, kmer_emb[:, 0], site_emb[:, 0])
    return out
